# Initial kernel scaffold; baseline (speedup 1.0000x reference)
import jax
import jax.numpy as jnp
from jax.experimental import pallas as pl


def kernel(raw_state, has_proprio, embodiment_idx, W_ph1, b_ph1, W_ph2, b_ph2, W_pr1, b_pr1, W_pr2, b_pr2, ln_g, ln_b, W_t1, b_t1, W_t2, b_t2):
    raise NotImplementedError("write your pallas kernel here")


# speedup vs baseline: 4.3863x; 4.3863x over previous
import functools

import jax
import jax.numpy as jnp
from jax.experimental import pallas as pl
from jax.experimental.pallas import tpu as pltpu

_B = 1024
_S = 64
_D = 1024
_H = 2048
_NE = 8
_R = 128
_T = _B // _R
_G = 2 * _NE
_NA = _T + _G - 1
_NB = _T + _NE - 1
_EPS = 1e-5


def _routing_tables(key16):
    ii_a = jnp.arange(_NA, dtype=jnp.int32)
    ii_b = jnp.arange(_NB, dtype=jnp.int32)

    def tables(key, ngroups, ni, ii):
        gids = jnp.arange(ngroups, dtype=jnp.int32)
        counts = jnp.sum((key[None, :] == gids[:, None]).astype(jnp.int32), axis=1)
        starts = jnp.cumsum(counts) - counts
        ends = starts + counts
        tlo = starts // _R
        thi = (ends + _R - 1) // _R
        ntiles = jnp.where(counts > 0, thi - tlo, 0)
        iend = jnp.cumsum(ntiles)
        total = iend[ngroups - 1]
        g = jnp.searchsorted(iend, ii, side='right').astype(jnp.int32)
        valid = ii < total
        g = jnp.minimum(g, ngroups - 1)
        first = iend[g] - ntiles[g]
        t = tlo[g] + (ii - first)
        t = jnp.clip(jnp.where(valid, t, _T - 1), 0, _T - 1)
        lo = starts[g]
        hi = jnp.where(valid, ends[g], 0)
        return g, t, lo, hi, valid

    key8 = key16 // 2
    ga, ta, loa, hia, va = tables(key16, _G, _NA, ii_a)
    gb, tb, lob, hib, vb = tables(key8, _NE, _NB, ii_b)

    emb_a = ga // 2
    par_a = ga % 2
    def hold_last(use, e, ii):
        enc = jnp.where(use, ii * _NE + e, -1)
        run = jax.lax.cummax(enc)
        return jnp.where(run >= 0, run % _NE, 0).astype(jnp.int32)

    phe = hold_last((par_a == 0) & va, emb_a, ii_a)
    pre = hold_last((par_a == 1) & va, emb_a, ii_a)
    eb = hold_last(vb, gb, ii_b)

    to32 = lambda x: x.astype(jnp.int32)
    return (to32(ta), to32(loa), to32(hia), to32(par_a), to32(va),
            to32(emb_a), phe, pre,
            to32(tb), to32(lob), to32(hib), to32(vb), eb)


def _gelu(x):
    return 0.5 * x * (1.0 + jax.lax.erf(x * 0.7071067811865476))


def _stage_a_body(s_t, s_lo, s_hi, s_par, s_valid, s_emb, s_phe, s_pre,
                  ordc_ref, x_ref,
                  wph1_ref, bph1_ref, wph2_ref, bph2_ref,
                  wpr1_ref, bpr1_ref, wpr2_ref, bpr2_ref,
                  lng_ref, lnb_ref, out_ref):
    i = pl.program_id(0)
    valid = s_valid[i] > 0
    par = s_par[i]

    def run(w1_ref, b1_ref, w2_ref, b2_ref):
        ordc = ordc_ref[0]
        onehot = (ordc == jax.lax.broadcasted_iota(jnp.int32, (_R, _B), 1)
                  ).astype(jnp.float32)
        xs = jnp.dot(onehot, x_ref[...], preferred_element_type=jnp.float32)
        h = _gelu(jnp.dot(xs, w1_ref[0], preferred_element_type=jnp.float32)
                  + b1_ref[0])
        y = jnp.dot(h, w2_ref[0], preferred_element_type=jnp.float32) + b2_ref[0]
        mu = jnp.mean(y, axis=1, keepdims=True)
        var = jnp.mean(jnp.square(y - mu), axis=1, keepdims=True)
        yn = (y - mu) * jax.lax.rsqrt(var + _EPS) * lng_ref[0] + lnb_ref[0]
        p = s_t[i] * _R + jax.lax.broadcasted_iota(jnp.int32, (_R, 1), 0)
        gmask = (p >= s_lo[i]) & (p < s_hi[i])
        out_ref[...] = jnp.where(gmask, yn, out_ref[...])

    @pl.when(valid & (par == 0))
    def _():
        run(wph1_ref, bph1_ref, wph2_ref, bph2_ref)

    @pl.when(valid & (par == 1))
    def _():
        run(wpr1_ref, bpr1_ref, wpr2_ref, bpr2_ref)


def _stage_b_body(s_t, s_lo, s_hi, s_valid, s_e,
                  ordr_ref, xin_ref, wt1_ref, bt1_ref, wt2_ref, bt2_ref,
                  out_ref):
    i = pl.program_id(0)

    @pl.when(i == 0)
    def _():
        out_ref[...] = jnp.zeros_like(out_ref)

    @pl.when(s_valid[i] > 0)
    def _():
        xs = xin_ref[...]
        h = _gelu(jnp.dot(xs, wt1_ref[0], preferred_element_type=jnp.float32)
                  + bt1_ref[0])
        y = jnp.dot(h, wt2_ref[0], preferred_element_type=jnp.float32) + bt2_ref[0]
        p = s_t[i] * _R + jax.lax.broadcasted_iota(jnp.int32, (_R, 1), 0)
        gmask = (p >= s_lo[i]) & (p < s_hi[i])
        ym = jnp.where(gmask, y, 0.0)
        ordr = ordr_ref[0]
        scat = (jax.lax.broadcasted_iota(jnp.int32, (_B, _R), 0) == ordr
                ).astype(jnp.float32)
        out_ref[...] += jnp.dot(scat, ym, preferred_element_type=jnp.float32)


@jax.jit
def kernel(raw_state, has_proprio, embodiment_idx, W_ph1, b_ph1, W_ph2, b_ph2,
           W_pr1, b_pr1, W_pr2, b_pr2, ln_g, ln_b, W_t1, b_t1, W_t2, b_t2):
    key16 = (embodiment_idx.astype(jnp.int32) * 2
             + has_proprio.astype(jnp.int32))
    order = jnp.argsort(key16).astype(jnp.int32)
    (ta, loa, hia, para, va, emba, phe, pre,
     tb, lob, hib, vb, eb) = _routing_tables(key16)

    ordc = order.reshape(_T, _R, 1)
    ordr = order.reshape(_T, 1, _R)

    mixed_ln = pl.pallas_call(
        _stage_a_body,
        grid_spec=pltpu.PrefetchScalarGridSpec(
            num_scalar_prefetch=8,
            grid=(_NA,),
            in_specs=[
                pl.BlockSpec((1, _R, 1),
                             lambda i, st, *_: (st[i], 0, 0)),
                pl.BlockSpec((_B, _S), lambda i, *_: (0, 0)),
                pl.BlockSpec((1, _S, _H),
                             lambda i, st, slo, shi, sp, sv, se, sphe, spre:
                             (sphe[i], 0, 0)),
                pl.BlockSpec((1, 1, _H),
                             lambda i, st, slo, shi, sp, sv, se, sphe, spre:
                             (sphe[i], 0, 0)),
                pl.BlockSpec((1, _H, _D),
                             lambda i, st, slo, shi, sp, sv, se, sphe, spre:
                             (sphe[i], 0, 0)),
                pl.BlockSpec((1, 1, _D),
                             lambda i, st, slo, shi, sp, sv, se, sphe, spre:
                             (sphe[i], 0, 0)),
                pl.BlockSpec((1, _S, _H),
                             lambda i, st, slo, shi, sp, sv, se, sphe, spre:
                             (spre[i], 0, 0)),
                pl.BlockSpec((1, 1, _H),
                             lambda i, st, slo, shi, sp, sv, se, sphe, spre:
                             (spre[i], 0, 0)),
                pl.BlockSpec((1, _H, _D),
                             lambda i, st, slo, shi, sp, sv, se, sphe, spre:
                             (spre[i], 0, 0)),
                pl.BlockSpec((1, 1, _D),
                             lambda i, st, slo, shi, sp, sv, se, sphe, spre:
                             (spre[i], 0, 0)),
                pl.BlockSpec((1, 1, _D),
                             lambda i, st, slo, shi, sp, sv, se, sphe, spre:
                             (se[i], 0, 0)),
                pl.BlockSpec((1, 1, _D),
                             lambda i, st, slo, shi, sp, sv, se, sphe, spre:
                             (se[i], 0, 0)),
            ],
            out_specs=pl.BlockSpec((_R, _D), lambda i, st, *_: (st[i], 0)),
        ),
        out_shape=jax.ShapeDtypeStruct((_B, _D), jnp.float32),
    )(ta, loa, hia, para, va, emba, phe, pre,
      ordc, raw_state,
      W_ph1, b_ph1[:, None, :], W_ph2, b_ph2[:, None, :],
      W_pr1, b_pr1[:, None, :], W_pr2, b_pr2[:, None, :],
      ln_g[:, None, :], ln_b[:, None, :])

    out = pl.pallas_call(
        _stage_b_body,
        grid_spec=pltpu.PrefetchScalarGridSpec(
            num_scalar_prefetch=5,
            grid=(_NB,),
            in_specs=[
                pl.BlockSpec((1, 1, _R),
                             lambda i, st, *_: (st[i], 0, 0)),
                pl.BlockSpec((_R, _D), lambda i, st, *_: (st[i], 0)),
                pl.BlockSpec((1, _D, _H),
                             lambda i, st, slo, shi, sv, se: (se[i], 0, 0)),
                pl.BlockSpec((1, 1, _H),
                             lambda i, st, slo, shi, sv, se: (se[i], 0, 0)),
                pl.BlockSpec((1, _H, _D),
                             lambda i, st, slo, shi, sv, se: (se[i], 0, 0)),
                pl.BlockSpec((1, 1, _D),
                             lambda i, st, slo, shi, sv, se: (se[i], 0, 0)),
            ],
            out_specs=pl.BlockSpec((_B, _D), lambda i, *_: (0, 0)),
        ),
        out_shape=jax.ShapeDtypeStruct((_B, _D), jnp.float32),
    )(tb, lob, hib, vb, eb,
      ordr, mixed_ln, W_t1, b_t1[:, None, :], W_t2, b_t2[:, None, :])

    return out[:, None, :]
